# baseline (device time: 79471 ns/iter reference)
import jax
import jax.numpy as jnp
from jax import lax
from jax.experimental import pallas as pl
from jax.experimental.pallas import tpu as pltpu

B = 16
NB = 128
BS = 16
H = 16
D = 64
P_LOCAL = 128
T_LOCAL = P_LOCAL * BS
SCALE = D ** -0.5
NEG = -1e30


def kernel(Q, K, V, bt, lens):
    q = Q.reshape(B, H, D)
    k = K.reshape(T_LOCAL, H, D)
    v = V.reshape(T_LOCAL, H, D)
    lens2 = lens.reshape(B, 1)

    def body(q_ref, k_ref, v_ref, bt_ref, lens_ref, out_ref,
             o_send, st_send, o_recv, st_recv, send_sems, recv_sems):
        my_x = lax.axis_index("x")
        my_y = lax.axis_index("y")
        peer = (1 - my_x, my_y)

        barrier = pltpu.get_barrier_semaphore()
        pl.semaphore_signal(barrier, inc=1, device_id=peer,
                            device_id_type=pl.DeviceIdType.MESH)
        pl.semaphore_wait(barrier, 1)

        x_off = my_x * P_LOCAL
        bt_arr = bt_ref[...]
        lens_arr = lens_ref[...]
        slot = lax.broadcasted_iota(jnp.int32, (B, NB, P_LOCAL), 1)
        page = lax.broadcasted_iota(jnp.int32, (B, NB, P_LOCAL), 2)
        hit = (bt_arr[:, :, None] == page + x_off) & (
            slot < lens_arr[:, :, None])
        w = jnp.sum(hit.astype(jnp.float32), axis=1)

        tpage = lax.broadcasted_iota(jnp.int32, (P_LOCAL, T_LOCAL), 1) // BS
        prow = lax.broadcasted_iota(jnp.int32, (P_LOCAL, T_LOCAL), 0)
        expand = (tpage == prow).astype(jnp.bfloat16)
        w_tok = lax.dot_general(
            w.astype(jnp.bfloat16), expand,
            (((1,), (0,)), ((), ())),
            preferred_element_type=jnp.float32,
        )

        dimnums_nt = (((1,), (1,)), ((), ()))
        m_list, l_list, o_list = [], [], []
        for h in range(H):
            qh = q_ref[:, h, :].astype(jnp.bfloat16)
            kh = k_ref[:, h, :].astype(jnp.bfloat16)
            s = lax.dot_general(
                qh, kh, dimnums_nt, preferred_element_type=jnp.float32,
            ) * SCALE
            s = jnp.where(w_tok > 0, s, NEG)
            m_h = jnp.max(s, axis=1, keepdims=True)
            p_un = jnp.exp(s - m_h) * w_tok
            l_h = jnp.sum(p_un, axis=1, keepdims=True)
            vh = v_ref[:, h, :].astype(jnp.bfloat16)
            o_h = lax.dot_general(
                p_un.astype(jnp.bfloat16), vh,
                (((1,), (0,)), ((), ())),
                preferred_element_type=jnp.float32,
            )
            m_list.append(m_h)
            l_list.append(l_h)
            o_list.append(o_h)

        m_loc = jnp.concatenate(m_list, axis=1)
        l_loc = jnp.concatenate(l_list, axis=1)
        o_loc = jnp.stack(o_list, axis=1)

        o_send[...] = o_loc
        st_send[0] = m_loc
        st_send[1] = l_loc
        rdma_o = pltpu.make_async_remote_copy(
            src_ref=o_send, dst_ref=o_recv,
            send_sem=send_sems.at[0], recv_sem=recv_sems.at[0],
            device_id=peer, device_id_type=pl.DeviceIdType.MESH,
        )
        rdma_st = pltpu.make_async_remote_copy(
            src_ref=st_send, dst_ref=st_recv,
            send_sem=send_sems.at[1], recv_sem=recv_sems.at[1],
            device_id=peer, device_id_type=pl.DeviceIdType.MESH,
        )
        rdma_o.start()
        rdma_st.start()
        rdma_o.wait()
        rdma_st.wait()

        m_p = st_recv[0]
        l_p = st_recv[1]
        o_p = o_recv[...]
        m_new = jnp.maximum(m_loc, m_p)
        a = jnp.exp(m_loc - m_new)
        c = jnp.exp(m_p - m_new)
        l_new = l_loc * a + l_p * c
        out_ref[...] = (
            o_loc * a[:, :, None] + o_p * c[:, :, None]
        ) / l_new[:, :, None]

    out = pl.pallas_call(
        body,
        out_shape=jax.ShapeDtypeStruct((B, H, D), jnp.float32),
        in_specs=[
            pl.BlockSpec(memory_space=pltpu.VMEM),
            pl.BlockSpec(memory_space=pltpu.VMEM),
            pl.BlockSpec(memory_space=pltpu.VMEM),
            pl.BlockSpec(memory_space=pltpu.VMEM),
            pl.BlockSpec(memory_space=pltpu.VMEM),
        ],
        out_specs=pl.BlockSpec(memory_space=pltpu.VMEM),
        scratch_shapes=[
            pltpu.VMEM((B, H, D), jnp.float32),
            pltpu.VMEM((2, B, H), jnp.float32),
            pltpu.VMEM((B, H, D), jnp.float32),
            pltpu.VMEM((2, B, H), jnp.float32),
            pltpu.SemaphoreType.DMA((2,)),
            pltpu.SemaphoreType.DMA((2,)),
        ],
        compiler_params=pltpu.CompilerParams(collective_id=0),
    )(q, k, v, bt, lens2)

    return out.reshape(B, 1, H, D)


# device time: 35254 ns/iter; 2.2542x vs baseline; 2.2542x over previous
import jax
import jax.numpy as jnp
from jax import lax
from jax.experimental import pallas as pl
from jax.experimental.pallas import tpu as pltpu

B = 16
NB = 128
BS = 16
H = 16
D = 64
P_LOCAL = 128
T_LOCAL = P_LOCAL * BS
SCALE = D ** -0.5
NEG = -1e30


def kernel(Q, K, V, bt, lens):
    q = Q.reshape(B, H, D).astype(jnp.bfloat16).swapaxes(0, 1)
    k = K.reshape(T_LOCAL, H, D).astype(jnp.bfloat16).swapaxes(0, 1)
    v = V.reshape(T_LOCAL, H, D).astype(jnp.bfloat16).swapaxes(0, 1)
    lens2 = lens.reshape(B, 1)

    def body(q_ref, k_ref, v_ref, bt_ref, lens_ref, out_ref,
             w_tok_ref, o_send, st_send, o_recv, st_recv,
             send_sems, recv_sems):
        h = pl.program_id(0)
        my_x = lax.axis_index("x")
        my_y = lax.axis_index("y")
        peer = (1 - my_x, my_y)

        @pl.when(h == 0)
        def _prologue():
            barrier = pltpu.get_barrier_semaphore()
            pl.semaphore_signal(barrier, inc=1, device_id=peer,
                                device_id_type=pl.DeviceIdType.MESH)
            pl.semaphore_wait(barrier, 1)

            x_off = my_x * P_LOCAL
            bt_arr = bt_ref[...]
            lens_arr = lens_ref[...]
            slot = lax.broadcasted_iota(jnp.int32, (B, NB, P_LOCAL), 1)
            page = lax.broadcasted_iota(jnp.int32, (B, NB, P_LOCAL), 2)
            hit = (bt_arr[:, :, None] == page + x_off) & (
                slot < lens_arr[:, :, None])
            w = jnp.sum(hit.astype(jnp.float32), axis=1)

            tpage = lax.broadcasted_iota(
                jnp.int32, (P_LOCAL, T_LOCAL), 1) // BS
            prow = lax.broadcasted_iota(jnp.int32, (P_LOCAL, T_LOCAL), 0)
            expand = (tpage == prow).astype(jnp.bfloat16)
            w_tok_ref[...] = lax.dot_general(
                w.astype(jnp.bfloat16), expand,
                (((1,), (0,)), ((), ())),
                preferred_element_type=jnp.float32,
            )

        w_tok = w_tok_ref[...]
        qh = q_ref[...]
        kh = k_ref[...]
        s = lax.dot_general(
            qh, kh, (((1,), (1,)), ((), ())),
            preferred_element_type=jnp.float32,
        ) * SCALE
        s = jnp.where(w_tok > 0, s, NEG)
        m_h = jnp.max(s, axis=1, keepdims=True)
        p_un = jnp.exp(s - m_h) * w_tok
        l_h = jnp.sum(p_un, axis=1, keepdims=True)
        vh = v_ref[...]
        o_h = lax.dot_general(
            p_un.astype(jnp.bfloat16), vh,
            (((1,), (0,)), ((), ())),
            preferred_element_type=jnp.float32,
        )

        o_send[pl.ds(h, 1)] = o_h[None, :, :]
        st_send[pl.ds(h, 1)] = jnp.concatenate([m_h, l_h], axis=1)[None]

        @pl.when(h == H - 1)
        def _epilogue():
            rdma_o = pltpu.make_async_remote_copy(
                src_ref=o_send, dst_ref=o_recv,
                send_sem=send_sems.at[0], recv_sem=recv_sems.at[0],
                device_id=peer, device_id_type=pl.DeviceIdType.MESH,
            )
            rdma_st = pltpu.make_async_remote_copy(
                src_ref=st_send, dst_ref=st_recv,
                send_sem=send_sems.at[1], recv_sem=recv_sems.at[1],
                device_id=peer, device_id_type=pl.DeviceIdType.MESH,
            )
            rdma_o.start()
            rdma_st.start()
            rdma_o.wait()
            rdma_st.wait()

            m_loc = st_send[:, :, 0:1]
            l_loc = st_send[:, :, 1:2]
            o_loc = o_send[...]
            m_p = st_recv[:, :, 0:1]
            l_p = st_recv[:, :, 1:2]
            o_p = o_recv[...]
            m_new = jnp.maximum(m_loc, m_p)
            a = jnp.exp(m_loc - m_new)
            c = jnp.exp(m_p - m_new)
            l_new = l_loc * a + l_p * c
            out_ref[...] = (o_loc * a + o_p * c) / l_new

    out = pl.pallas_call(
        body,
        grid=(H,),
        out_shape=jax.ShapeDtypeStruct((H, B, D), jnp.float32),
        in_specs=[
            pl.BlockSpec((None, B, D), lambda h: (h, 0, 0)),
            pl.BlockSpec((None, T_LOCAL, D), lambda h: (h, 0, 0)),
            pl.BlockSpec((None, T_LOCAL, D), lambda h: (h, 0, 0)),
            pl.BlockSpec((B, NB), lambda h: (0, 0)),
            pl.BlockSpec((B, 1), lambda h: (0, 0)),
        ],
        out_specs=pl.BlockSpec((H, B, D), lambda h: (0, 0, 0)),
        scratch_shapes=[
            pltpu.VMEM((B, T_LOCAL), jnp.float32),
            pltpu.VMEM((H, B, D), jnp.float32),
            pltpu.VMEM((H, B, 2), jnp.float32),
            pltpu.VMEM((H, B, D), jnp.float32),
            pltpu.VMEM((H, B, 2), jnp.float32),
            pltpu.SemaphoreType.DMA((2,)),
            pltpu.SemaphoreType.DMA((2,)),
        ],
        compiler_params=pltpu.CompilerParams(collective_id=0),
    )(q, k, v, bt, lens2)

    return out.swapaxes(0, 1).reshape(B, 1, H, D)
